# baseline (speedup 1.0000x reference)
import functools

import jax
import jax.numpy as jnp
from jax import lax
from jax.experimental import pallas as pl
from jax.experimental.pallas import tpu as pltpu
from jax.experimental.pallas import tpu_sc as plsc

_BATCH = 16384
_EMBED = 32
_BN_EPS = 1e-3

_NC = 2
_NS = 16
_NW = _NC * _NS
_BPW = _BATCH // _NW
_CHUNK = 128
_NPASS = _BPW // _CHUNK
_GPP = _CHUNK // 16


def _sc_kernel_body(uidx_hbm, cidx_hbm, ut_hbm, ct_hbm, sc_hbm, be_hbm,
                    out_hbm,
                    uidx_v, cidx_v, ublk_v, cblk_v, ubuf, cbuf,
                    sc_v, be_v, out_v, sem):
    wid = lax.axis_index("s") * _NC + lax.axis_index("c")

    pltpu.sync_copy(uidx_hbm.at[pl.ds(wid, 1)], uidx_v)
    pltpu.sync_copy(cidx_hbm.at[pl.ds(wid, 1)], cidx_v)
    pltpu.sync_copy(sc_hbm, sc_v)
    pltpu.sync_copy(be_hbm, be_v)

    for j in range(_BPW // 16):
        src = uidx_v[0, pl.ds(j * 16, 16)] >> 2
        ublk_v[j // 8, pl.ds((j % 8) * 16, 16)] = src
        srcc = cidx_v[0, pl.ds(j * 16, 16)] >> 2
        cblk_v[j // 8, pl.ds((j % 8) * 16, 16)] = srcc

    lane = lax.iota(jnp.int32, 16)
    s_half = [sc_v[pl.ds(0, 16)], sc_v[pl.ds(16, 16)]]
    b_half = [be_v[pl.ds(0, 16)], be_v[pl.ds(16, 16)]]

    for p in range(_NPASS):
        cu = pltpu.async_copy(ut_hbm.at[ublk_v.at[p]], ubuf, sem)
        cc = pltpu.async_copy(ct_hbm.at[cblk_v.at[p]], cbuf, sem)
        cu.wait()
        cc.wait()

        def group_body(g, carry, p=p):
            base = p * _CHUNK + g * 16
            pos = lane + g * 16
            usub = (uidx_v[0, pl.ds(base, 16)] & 3) * _EMBED
            csub = (cidx_v[0, pl.ds(base, 16)] & 3) * _EMBED
            acc = jnp.zeros((16,), jnp.float32)
            for d in range(_EMBED):
                u = plsc.load_gather(ubuf, [pos, usub + d])
                c = plsc.load_gather(cbuf, [pos, csub + d])
                s_d = s_half[d // 16][d % 16]
                b_d = b_half[d // 16][d % 16]
                acc = acc + (u * s_d + b_d) * (c * s_d + b_d)
            out_v[pl.ds(base, 16)] = acc
            return carry

        lax.fori_loop(0, _GPP, group_body, 0, unroll=False)

    pltpu.sync_copy(out_v, out_hbm.at[pl.ds(wid * _BPW, _BPW)])


@jax.jit
def _run(uidx, cidx, ut_b, ct_b, scale, beta):
    mesh = plsc.VectorSubcoreMesh(core_axis_name="c", subcore_axis_name="s")
    kern = functools.partial(
        pl.kernel,
        mesh=mesh,
        out_type=jax.ShapeDtypeStruct((_BATCH,), jnp.float32),
        scratch_types=[
            pltpu.VMEM((1, _BPW), jnp.int32),
            pltpu.VMEM((1, _BPW), jnp.int32),
            pltpu.VMEM((_NPASS, _CHUNK), jnp.int32),
            pltpu.VMEM((_NPASS, _CHUNK), jnp.int32),
            pltpu.VMEM((_CHUNK, 128), jnp.float32),
            pltpu.VMEM((_CHUNK, 128), jnp.float32),
            pltpu.VMEM((_EMBED,), jnp.float32),
            pltpu.VMEM((_EMBED,), jnp.float32),
            pltpu.VMEM((_BPW,), jnp.float32),
            pltpu.SemaphoreType.DMA,
        ],
        compiler_params=pltpu.CompilerParams(needs_layout_passes=False),
    )(_sc_kernel_body)
    return kern(uidx, cidx, ut_b, ct_b, scale, beta)


def kernel(user, content, user_table, content_table, gamma, beta):
    scale = gamma / jnp.sqrt(1.0 + _BN_EPS)
    uidx = user.reshape(_NW, _BPW).astype(jnp.int32)
    cidx = content.reshape(_NW, _BPW).astype(jnp.int32)
    ut_b = user_table.reshape(-1, 4 * _EMBED)
    ct_b = content_table.reshape(-1, 4 * _EMBED)
    out = _run(uidx, cidx, ut_b, ct_b, scale, beta)
    return out.reshape(_BATCH, 1)


# speedup vs baseline: 1.5607x; 1.5607x over previous
import functools

import jax
import jax.numpy as jnp
from jax import lax
from jax.experimental import pallas as pl
from jax.experimental.pallas import tpu as pltpu
from jax.experimental.pallas import tpu_sc as plsc

_BATCH = 16384
_EMBED = 32
_BN_EPS = 1e-3

_NC = 2
_NS = 16
_NW = _NC * _NS
_BPW = _BATCH // _NW
_GROUPS = _BPW // 16


def _sc_kernel_body(uidx_hbm, cidx_hbm, ut_hbm, ct_hbm, sc_hbm, be_hbm,
                    out_hbm,
                    uidx_v, cidx_v, ua_v, ub_v, ca_v, cb_v,
                    sc_v, be_v, out_v, sema, semb):
    wid = lax.axis_index("s") * _NC + lax.axis_index("c")

    pltpu.sync_copy(uidx_hbm.at[pl.ds(wid, 1)], uidx_v)
    pltpu.sync_copy(cidx_hbm.at[pl.ds(wid, 1)], cidx_v)
    pltpu.sync_copy(sc_hbm, sc_v)
    pltpu.sync_copy(be_hbm, be_v)

    lane = lax.iota(jnp.int32, 16)
    s0 = sc_v[pl.ds(0, 16)]
    s1 = sc_v[pl.ds(16, 16)]
    b0 = be_v[pl.ds(0, 16)]
    b1 = be_v[pl.ds(16, 16)]

    def fire(g, ubuf, cbuf, sem):
        uvec = uidx_v[0, pl.ds(g * 16, 16)]
        cvec = cidx_v[0, pl.ds(g * 16, 16)]
        for r in range(16):
            pltpu.async_copy(ut_hbm.at[pl.ds(uvec[r], 1)],
                             ubuf.at[pl.ds(r, 1)], sem)
            pltpu.async_copy(ct_hbm.at[pl.ds(cvec[r], 1)],
                             cbuf.at[pl.ds(r, 1)], sem)

    def drain(ubuf, cbuf, sem):
        for r in range(16):
            pltpu.make_async_copy(ut_hbm.at[pl.ds(0, 1)],
                                  ubuf.at[pl.ds(r, 1)], sem).wait()
            pltpu.make_async_copy(ct_hbm.at[pl.ds(0, 1)],
                                  cbuf.at[pl.ds(r, 1)], sem).wait()

    def compute(g, ubuf, cbuf):
        acc = jnp.zeros((16,), jnp.float32)
        for r in range(16):
            u0 = ubuf[r, pl.ds(0, 16)] * s0 + b0
            u1 = ubuf[r, pl.ds(16, 16)] * s1 + b1
            c0 = cbuf[r, pl.ds(0, 16)] * s0 + b0
            c1 = cbuf[r, pl.ds(16, 16)] * s1 + b1
            t = u0 * c0 + u1 * c1
            parts = [t[i] for i in range(16)]
            while len(parts) > 1:
                parts = [parts[i] + parts[i + 1]
                         for i in range(0, len(parts), 2)]
            acc = jnp.where(lane == r, parts[0], acc)
        out_v[pl.ds(g * 16, 16)] = acc

    fire(0, ua_v, ca_v, sema)

    def pair_body(k, carry):
        g0 = 2 * k
        fire(g0 + 1, ub_v, cb_v, semb)
        drain(ua_v, ca_v, sema)
        compute(g0, ua_v, ca_v)

        @pl.when(k < _GROUPS // 2 - 1)
        def _():
            fire(g0 + 2, ua_v, ca_v, sema)

        drain(ub_v, cb_v, semb)
        compute(g0 + 1, ub_v, cb_v)
        return carry

    lax.fori_loop(0, _GROUPS // 2, pair_body, 0, unroll=False)

    pltpu.sync_copy(out_v, out_hbm.at[pl.ds(wid * _BPW, _BPW)])


@jax.jit
def _run(uidx, cidx, user_table, content_table, scale, beta):
    mesh = plsc.VectorSubcoreMesh(core_axis_name="c", subcore_axis_name="s")
    kern = functools.partial(
        pl.kernel,
        mesh=mesh,
        out_type=jax.ShapeDtypeStruct((_BATCH,), jnp.float32),
        scratch_types=[
            pltpu.VMEM((1, _BPW), jnp.int32),
            pltpu.VMEM((1, _BPW), jnp.int32),
            pltpu.VMEM((16, _EMBED), jnp.float32),
            pltpu.VMEM((16, _EMBED), jnp.float32),
            pltpu.VMEM((16, _EMBED), jnp.float32),
            pltpu.VMEM((16, _EMBED), jnp.float32),
            pltpu.VMEM((_EMBED,), jnp.float32),
            pltpu.VMEM((_EMBED,), jnp.float32),
            pltpu.VMEM((_BPW,), jnp.float32),
            pltpu.SemaphoreType.DMA,
            pltpu.SemaphoreType.DMA,
        ],
    )(_sc_kernel_body)
    return kern(uidx, cidx, user_table, content_table, scale, beta)


def kernel(user, content, user_table, content_table, gamma, beta):
    scale = gamma / jnp.sqrt(1.0 + _BN_EPS)
    uidx = user.reshape(_NW, _BPW).astype(jnp.int32)
    cidx = content.reshape(_NW, _BPW).astype(jnp.int32)
    out = _run(uidx, cidx, user_table, content_table, scale, beta)
    return out.reshape(_BATCH, 1)
